# baseline (speedup 1.0000x reference)
import functools

import jax
import jax.numpy as jnp
from jax import lax
from jax.experimental import pallas as pl
from jax.experimental.pallas import tpu as pltpu
from jax.experimental.pallas import tpu_sc as plsc

D = 64
B = 16384
NC = 2
NS = 16
NW = NC * NS
BPW = B // NW
L = 16
NCH = BPW // L
TL = 128
NB = 14
K = 12

TILE_BYTES = D * TL * 4


def _gather_body(tabT, pos_s, fidx_s, newf_s, out, *refs):
    (idx_p, idx_f, idx_n, stk, colv, sem, sem_out) = refs

    c = lax.axis_index("c")
    s = lax.axis_index("s")
    wid = s * NC + c
    base = wid * BPW

    pltpu.sync_copy(pos_s.at[pl.ds(base, BPW)], idx_p)
    pltpu.sync_copy(fidx_s.at[pl.ds(base, BPW)], idx_f)
    pltpu.sync_copy(newf_s.at[pl.ds(base, BPW)], idx_n)

    iota = lax.broadcasted_iota(jnp.int32, (L,), 0)

    def enqueue(pos_j, nf_j, fx_j):
        col = pl.multiple_of((pos_j // TL) * TL, TL)
        slot = (fx_j - 1) % NB

        @pl.when(nf_j == 1)
        def _():
            pltpu.make_async_copy(
                tabT.at[:, pl.ds(col, TL)], stk.at[slot], sem).start()

    def consume(e_scal, pos_j, fx_j, oslot, waited):
        def wbody(_, w):
            pltpu.make_async_copy(
                tabT.at[:, pl.ds(0, TL)], stk.at[0], sem).wait()
            return w
        lax.fori_loop(0, fx_j - waited, wbody, 0)
        waited = jnp.maximum(waited, fx_j)
        q = jnp.full((L,), pos_j % TL, jnp.int32)
        slotv = jnp.full((L,), (fx_j - 1) % NB, jnp.int32)
        for g in range(4):
            vals = plsc.load_gather(stk, [slotv, g * L + iota, q])
            colv[oslot, pl.ds(g * L, L)] = vals
        return waited

    def flush(e_next, half):
        @pl.when(e_next >= 0)
        def _():
            dst = pl.multiple_of(base + e_next - 7, 8)
            pltpu.make_async_copy(
                colv.at[pl.ds(half * 8, 8), :],
                out.at[pl.ds(dst, 8), :], sem_out).start()

    def chunk_body(g, carry):
        waited, ppos, pfx = carry
        pos_v = idx_p[pl.ds(g * L, L)]
        fx_v = idx_f[pl.ds(g * L, L)]
        nf_v = idx_n[pl.ds(g * L, L)]
        for j in range(L):
            e = g * L + j - K
            lane = (j + L - K) % L
            spos = ppos if j < K else pos_v
            sfx = pfx if j < K else fx_v
            if j < K:
                waited = lax.cond(
                    e >= 0,
                    lambda w: _consume_wrap(e, spos[lane], sfx[lane],
                                            lane, w),
                    lambda w: w,
                    waited)
            else:
                waited = _consume_wrap(e, spos[lane], sfx[lane], lane,
                                       waited)
            if lane == 7:
                flush(e, 0)
            if lane == 15:
                flush(e, 1)
            enqueue(pos_v[j], nf_v[j], fx_v[j])
        return (waited, pos_v, fx_v)

    def _consume_wrap(e, pos_j, fx_j, oslot, waited):
        if oslot in (0, 8):
            @pl.when(e >= 16)
            def _():
                pltpu.make_async_copy(
                    colv.at[pl.ds(0, 8), :], out.at[pl.ds(0, 8), :],
                    sem_out).wait()
        return consume(e, pos_j, fx_j, oslot, waited)

    zero = jnp.int32(0)
    pos0 = idx_p[pl.ds(0, L)]
    waited, ppos, pfx = lax.fori_loop(
        0, NCH, chunk_body, (zero, pos0, pos0))

    for j in range(K):
        e = BPW - K + j
        lane = L - K + j
        waited = _consume_wrap(e, ppos[lane], pfx[lane], lane, waited)
        if lane == 7:
            flush(jnp.int32(e), 0)
        if lane == 15:
            flush(jnp.int32(e), 1)
    for _ in range(2):
        pltpu.make_async_copy(
            colv.at[pl.ds(0, 8), :], out.at[pl.ds(0, 8), :], sem_out).wait()


_gather_call = functools.partial(
    pl.kernel,
    out_type=jax.ShapeDtypeStruct((B, TL), jnp.float32),
    mesh=plsc.VectorSubcoreMesh(core_axis_name="c", subcore_axis_name="s"),
    compiler_params=pltpu.CompilerParams(
        use_tc_tiling_on_sc=True, needs_layout_passes=False),
    scratch_types=[
        pltpu.VMEM((BPW,), jnp.int32),
        pltpu.VMEM((BPW,), jnp.int32),
        pltpu.VMEM((BPW,), jnp.int32),
        pltpu.VMEM((NB, D, TL), jnp.float32),
        pltpu.VMEM((L, TL), jnp.float32),
        pltpu.SemaphoreType.DMA,
        pltpu.SemaphoreType.DMA,
    ],
)(_gather_body)


def _combine_body(ucols, vcols, rank_u, rank_v, ubt, ibt, uidx, iidx,
                  part_out, bsum_out,
                  ru0, ru1, rv0, rv1, rows_u, rows_i,
                  idx_u, idx_i, bu, bi, accv,
                  sem_u, sem_i, sem_b):
    c = lax.axis_index("c")
    s = lax.axis_index("s")
    wid = s * NC + c
    base = wid * BPW
    H = BPW // 2

    pltpu.sync_copy(rank_u.at[pl.ds(base, H)], ru0)
    pltpu.sync_copy(rank_u.at[pl.ds(base + H, H)], ru1)
    pltpu.sync_copy(rank_v.at[pl.ds(base, H)], rv0)
    pltpu.sync_copy(rank_v.at[pl.ds(base + H, H)], rv1)
    pltpu.sync_copy(uidx.at[pl.ds(base, BPW)], idx_u)
    pltpu.sync_copy(iidx.at[pl.ds(base, BPW)], idx_i)
    cp_bu = pltpu.async_copy(ubt.at[idx_u], bu, sem_b)
    cp_bi = pltpu.async_copy(ibt.at[idx_i], bi, sem_b)

    zero = jnp.zeros((L,), jnp.float32)
    accs = [zero, zero, zero, zero]
    for h, (ru, rv) in enumerate(((ru0, rv0), (ru1, rv1))):
        cu = pltpu.async_copy(ucols.at[ru], rows_u, sem_u)
        cv = pltpu.async_copy(vcols.at[rv], rows_i, sem_i)
        cu.wait()
        cv.wait()

        def body(r, acc):
            a0, a1, a2, a3 = acc
            a0 = a0 + rows_u[r, pl.ds(0, L)] * rows_i[r, pl.ds(0, L)]
            a1 = a1 + rows_u[r, pl.ds(L, L)] * rows_i[r, pl.ds(L, L)]
            a2 = a2 + rows_u[r, pl.ds(2 * L, L)] * rows_i[r, pl.ds(2 * L, L)]
            a3 = a3 + rows_u[r, pl.ds(3 * L, L)] * rows_i[r, pl.ds(3 * L, L)]
            return (a0, a1, a2, a3)

        accs = list(lax.fori_loop(0, H, body, tuple(accs)))

    accv[...] = (accs[0] + accs[1]) + (accs[2] + accs[3])
    pltpu.sync_copy(accv, part_out.at[pl.ds(wid * L, L)])

    cp_bu.wait()
    cp_bi.wait()
    for j in range(BPW // L):
        sl = pl.ds(j * L, L)
        bu[sl] = bu[sl] + bi[sl]
    pltpu.sync_copy(bu, bsum_out.at[pl.ds(base, BPW)])


_combine_call = functools.partial(
    pl.kernel,
    out_type=(
        jax.ShapeDtypeStruct((NW * L,), jnp.float32),
        jax.ShapeDtypeStruct((B,), jnp.float32),
    ),
    mesh=plsc.VectorSubcoreMesh(core_axis_name="c", subcore_axis_name="s"),
    compiler_params=pltpu.CompilerParams(use_tc_tiling_on_sc=False),
    scratch_types=(
        [pltpu.VMEM((BPW // 2,), jnp.int32)] * 4
        + [pltpu.VMEM((BPW // 2, TL), jnp.float32)] * 2
        + [pltpu.VMEM((BPW,), jnp.int32)] * 2
        + [pltpu.VMEM((BPW,), jnp.float32)] * 2
        + [pltpu.VMEM((L,), jnp.float32)]
        + [pltpu.SemaphoreType.DMA] * 3
    ),
)(_combine_body)


def _finish_body(part_ref, bsum_ref, out_ref):
    scalar = jnp.sum(part_ref[...])
    out_ref[...] = jnp.maximum(bsum_ref[...] + scalar, 0.0)


def _prep(pos, bb):
    srt, perm = lax.sort_key_val(pos, bb)
    col = srt // TL
    prev = jnp.concatenate([col[:1] - 1, col[:-1]])
    newf = ((col != prev) | (bb % BPW == 0)).astype(jnp.int32)
    fidx = jnp.cumsum(newf.reshape(NW, BPW), axis=1).reshape(-1)
    fidx = fidx.astype(jnp.int32)
    rank = jnp.zeros((B,), jnp.int32).at[perm].set(bb, mode="promise_in_bounds",
                                                   unique_indices=True)
    return srt, fidx, newf, rank


def kernel(user_emb, user_bias_tbl, item_emb, item_bias_tbl, inputs):
    idx = inputs.astype(jnp.int32).T
    uidx, iidx = idx[0], idx[1]
    bb = lax.broadcasted_iota(jnp.int32, (B,), 0)
    spos_u, fidx_u, newf_u, rank_u = _prep(uidx, bb)
    spos_v, fidx_v, newf_v, rank_v = _prep(iidx, bb)
    ucols = _gather_call(user_emb.T, spos_u, fidx_u, newf_u)
    vcols = _gather_call(item_emb.T, spos_v, fidx_v, newf_v)
    partials, bsum = _combine_call(
        ucols, vcols, rank_u, rank_v,
        user_bias_tbl.T.reshape(-1), item_bias_tbl.T.reshape(-1),
        uidx, iidx)
    out = pl.pallas_call(
        _finish_body,
        out_shape=jax.ShapeDtypeStruct((B // 128, 128), jnp.float32),
    )(partials.reshape(NW, L), bsum.reshape(B // 128, 128))
    return out.reshape(B, 1)


# speedup vs baseline: 1.2339x; 1.2339x over previous
import functools

import jax
import jax.numpy as jnp
from jax import lax
from jax.experimental import pallas as pl
from jax.experimental.pallas import tpu as pltpu
from jax.experimental.pallas import tpu_sc as plsc

D = 64
B = 16384
NC = 2
NS = 16
NW = NC * NS
BPW = B // NW
L = 16
NCH = BPW // L
TL = 128
NB = 14
K = 12

TILE_BYTES = D * TL * 4


def _gather_body(tabT, pos_s, fidx_s, newf_s, bperm, out, *refs):
    (idx_p, idx_f, idx_n, idx_b, stk, colv, sem, sem_out) = refs

    c = lax.axis_index("c")
    s = lax.axis_index("s")
    wid = s * NC + c
    base = wid * BPW

    pltpu.sync_copy(pos_s.at[pl.ds(base, BPW)], idx_p)
    pltpu.sync_copy(fidx_s.at[pl.ds(base, BPW)], idx_f)
    pltpu.sync_copy(newf_s.at[pl.ds(base, BPW)], idx_n)
    pltpu.sync_copy(bperm.at[wid], idx_b)

    iota = lax.broadcasted_iota(jnp.int32, (L,), 0)

    def enqueue(pos_j, nf_j, fx_j):
        col = pl.multiple_of((pos_j // TL) * TL, TL)
        slot = (fx_j - 1) % NB

        @pl.when(nf_j == 1)
        def _():
            pltpu.make_async_copy(
                tabT.at[:, pl.ds(col, TL)], stk.at[slot], sem).start()

    def consume(e_scal, pos_j, fx_j, oslot, waited):
        def wbody(_, w):
            pltpu.make_async_copy(
                tabT.at[:, pl.ds(0, TL)], stk.at[0], sem).wait()
            return w
        lax.fori_loop(0, fx_j - waited, wbody, 0)
        waited = jnp.maximum(waited, fx_j)
        q = jnp.full((L,), pos_j % TL, jnp.int32)
        slotv = jnp.full((L,), (fx_j - 1) % NB, jnp.int32)
        for g in range(4):
            vals = plsc.load_gather(stk, [slotv, g * L + iota, q])
            colv[oslot, pl.ds(g * L, L)] = vals
        return waited

    def flush(e_last):
        @pl.when(e_last >= 0)
        def _():
            pltpu.make_async_copy(
                colv, out.at[idx_b.at[e_last // L]], sem_out).start()

    def chunk_body(g, carry):
        waited, ppos, pfx = carry
        pos_v = idx_p[pl.ds(g * L, L)]
        fx_v = idx_f[pl.ds(g * L, L)]
        nf_v = idx_n[pl.ds(g * L, L)]
        for j in range(L):
            e = g * L + j - K
            lane = (j + L - K) % L
            spos = ppos if j < K else pos_v
            sfx = pfx if j < K else fx_v
            if j < K:
                waited = lax.cond(
                    e >= 0,
                    lambda w: _consume_wrap(e, spos[lane], sfx[lane],
                                            lane, w),
                    lambda w: w,
                    waited)
            else:
                waited = _consume_wrap(e, spos[lane], sfx[lane], lane,
                                       waited)
            if lane == 15:
                flush(e)
            enqueue(pos_v[j], nf_v[j], fx_v[j])
        return (waited, pos_v, fx_v)

    def _consume_wrap(e, pos_j, fx_j, oslot, waited):
        if oslot == 0:
            @pl.when(e >= L)
            def _():
                pltpu.make_async_copy(
                    colv, out.at[pl.ds(0, L)], sem_out).wait()
        return consume(e, pos_j, fx_j, oslot, waited)

    zero = jnp.int32(0)
    pos0 = idx_p[pl.ds(0, L)]
    waited, ppos, pfx = lax.fori_loop(
        0, NCH, chunk_body, (zero, pos0, pos0))

    for j in range(K):
        e = BPW - K + j
        lane = L - K + j
        waited = _consume_wrap(e, ppos[lane], pfx[lane], lane, waited)
        if lane == 15:
            flush(jnp.int32(e))
    pltpu.make_async_copy(colv, out.at[pl.ds(0, L)], sem_out).wait()


_gather_call = functools.partial(
    pl.kernel,
    out_type=jax.ShapeDtypeStruct((B, TL), jnp.float32),
    mesh=plsc.VectorSubcoreMesh(core_axis_name="c", subcore_axis_name="s"),
    compiler_params=pltpu.CompilerParams(
        use_tc_tiling_on_sc=True, needs_layout_passes=False),
    scratch_types=[
        pltpu.VMEM((BPW,), jnp.int32),
        pltpu.VMEM((BPW,), jnp.int32),
        pltpu.VMEM((BPW,), jnp.int32),
        pltpu.VMEM((NCH, L), jnp.int32),
        pltpu.VMEM((NB, D, TL), jnp.float32),
        pltpu.VMEM((L, TL), jnp.float32),
        pltpu.SemaphoreType.DMA,
        pltpu.SemaphoreType.DMA,
    ],
)(_gather_body)


def _combine_body(ucols, vcols, ubt, ibt, uidx, iidx,
                  part_out, bsum_out,
                  rows_u, rows_i,
                  idx_u, idx_i, bu, bi, accv,
                  sem_u, sem_i, sem_b):
    c = lax.axis_index("c")
    s = lax.axis_index("s")
    wid = s * NC + c
    base = wid * BPW
    H = BPW // 2

    pltpu.sync_copy(uidx.at[pl.ds(base, BPW)], idx_u)
    pltpu.sync_copy(iidx.at[pl.ds(base, BPW)], idx_i)
    cp_bu = pltpu.async_copy(ubt.at[idx_u], bu, sem_b)
    cp_bi = pltpu.async_copy(ibt.at[idx_i], bi, sem_b)

    zero = jnp.zeros((L,), jnp.float32)
    accs = [zero, zero, zero, zero]
    for h in range(2):
        cu = pltpu.async_copy(
            ucols.at[pl.ds(base + h * H, H), :], rows_u, sem_u)
        cv = pltpu.async_copy(
            vcols.at[pl.ds(base + h * H, H), :], rows_i, sem_i)
        cu.wait()
        cv.wait()

        def body(r, acc):
            a0, a1, a2, a3 = acc
            a0 = a0 + rows_u[r, pl.ds(0, L)] * rows_i[r, pl.ds(0, L)]
            a1 = a1 + rows_u[r, pl.ds(L, L)] * rows_i[r, pl.ds(L, L)]
            a2 = a2 + rows_u[r, pl.ds(2 * L, L)] * rows_i[r, pl.ds(2 * L, L)]
            a3 = a3 + rows_u[r, pl.ds(3 * L, L)] * rows_i[r, pl.ds(3 * L, L)]
            return (a0, a1, a2, a3)

        accs = list(lax.fori_loop(0, H, body, tuple(accs)))

    accv[...] = (accs[0] + accs[1]) + (accs[2] + accs[3])
    pltpu.sync_copy(accv, part_out.at[pl.ds(wid * L, L)])

    cp_bu.wait()
    cp_bi.wait()
    for j in range(BPW // L):
        sl = pl.ds(j * L, L)
        bu[sl] = bu[sl] + bi[sl]
    pltpu.sync_copy(bu, bsum_out.at[pl.ds(base, BPW)])


_combine_call = functools.partial(
    pl.kernel,
    out_type=(
        jax.ShapeDtypeStruct((NW * L,), jnp.float32),
        jax.ShapeDtypeStruct((B,), jnp.float32),
    ),
    mesh=plsc.VectorSubcoreMesh(core_axis_name="c", subcore_axis_name="s"),
    compiler_params=pltpu.CompilerParams(use_tc_tiling_on_sc=False),
    scratch_types=(
        [pltpu.VMEM((BPW // 2, TL), jnp.float32)] * 2
        + [pltpu.VMEM((BPW,), jnp.int32)] * 2
        + [pltpu.VMEM((BPW,), jnp.float32)] * 2
        + [pltpu.VMEM((L,), jnp.float32)]
        + [pltpu.SemaphoreType.DMA] * 3
    ),
)(_combine_body)


def _finish_body(part_ref, bsum_ref, out_ref):
    scalar = jnp.sum(part_ref[...])
    out_ref[...] = jnp.maximum(bsum_ref[...] + scalar, 0.0)


def _prep(pos, bb):
    srt, perm = lax.sort_key_val(pos, bb)
    col = srt // TL
    prev = jnp.concatenate([col[:1] - 1, col[:-1]])
    newf = ((col != prev) | (bb % BPW == 0)).astype(jnp.int32)
    fidx = jnp.cumsum(newf.reshape(NW, BPW), axis=1).reshape(-1)
    fidx = fidx.astype(jnp.int32)
    return srt, fidx, newf, perm.reshape(NW, NCH, L)


def kernel(user_emb, user_bias_tbl, item_emb, item_bias_tbl, inputs):
    idx = inputs.astype(jnp.int32).T
    uidx, iidx = idx[0], idx[1]
    bb = lax.broadcasted_iota(jnp.int32, (B,), 0)
    spos_u, fidx_u, newf_u, perm_u = _prep(uidx, bb)
    spos_v, fidx_v, newf_v, perm_v = _prep(iidx, bb)
    ucols = _gather_call(user_emb.T, spos_u, fidx_u, newf_u, perm_u)
    vcols = _gather_call(item_emb.T, spos_v, fidx_v, newf_v, perm_v)
    partials, bsum = _combine_call(
        ucols, vcols,
        user_bias_tbl.T.reshape(-1), item_bias_tbl.T.reshape(-1),
        uidx, iidx)
    out = pl.pallas_call(
        _finish_body,
        out_shape=jax.ShapeDtypeStruct((B // 128, 128), jnp.float32),
    )(partials.reshape(NW, L), bsum.reshape(B // 128, 128))
    return out.reshape(B, 1)
